# Initial kernel scaffold; baseline (speedup 1.0000x reference)
import functools

import jax
import jax.numpy as jnp
from jax import lax
from jax.experimental import pallas as pl
from jax.experimental.pallas import tpu as pltpu
from jax.experimental.pallas import tpu_sc as plsc

EMBEDDING_DIM = 64

_info = plsc.get_sparse_core_info()
_NC, _NS = _info.num_cores, _info.num_subcores
_NW = _NC * _NS

_SUB = 128
_CHUNK = 512
_KSUB = _CHUNK // _SUB


def _gather_impl(n_rows, table, idx2d):
    n_per_w = n_rows // _NW
    n_chunks = n_per_w // _CHUNK
    mesh = plsc.VectorSubcoreMesh(core_axis_name="c", subcore_axis_name="s")

    @functools.partial(
        pl.kernel,
        mesh=mesh,
        out_type=jax.ShapeDtypeStruct((n_rows, EMBEDDING_DIM), jnp.float32),
        scratch_types=[
            pltpu.VMEM((_KSUB, _SUB), jnp.int32),
            pltpu.VMEM((_CHUNK, EMBEDDING_DIM), jnp.float32),
            pltpu.SemaphoreType.DMA,
        ],
    )
    def k(table_hbm, idx_hbm, out_hbm, idx_v, rows_v, sem):
        wid = lax.axis_index("s") * _NC + lax.axis_index("c")
        w_base = wid * n_per_w

        def body(g, carry):
            base = w_base + g * _CHUNK
            pltpu.sync_copy(idx_hbm.at[pl.ds(base // _SUB, _KSUB)], idx_v)
            copies = [
                pltpu.async_copy(
                    table_hbm.at[idx_v.at[j]],
                    rows_v.at[pl.ds(j * _SUB, _SUB)],
                    sem,
                )
                for j in range(_KSUB)
            ]
            for c in copies:
                c.wait()
            pltpu.sync_copy(rows_v, out_hbm.at[pl.ds(base, _CHUNK)])
            return carry

        lax.fori_loop(0, n_chunks, body, 0)

    return k(table, idx2d)


def kernel(inputs, table):
    b, t = inputs.shape
    n_rows = b * t
    idx2d = inputs.reshape(n_rows // _SUB, _SUB)
    out = _gather_impl(n_rows, table, idx2d)
    return out.reshape(b, t, EMBEDDING_DIM)


# speedup vs baseline: 1.6254x; 1.6254x over previous
import functools

import jax
import jax.numpy as jnp
from jax import lax
from jax.experimental import pallas as pl
from jax.experimental.pallas import tpu as pltpu
from jax.experimental.pallas import tpu_sc as plsc

EMBEDDING_DIM = 64

_info = plsc.get_sparse_core_info()
_NC, _NS = _info.num_cores, _info.num_subcores
_NW = _NC * _NS

_SUB = 128
_C2 = 512
_KSUB = _C2 // _SUB


def _gather_impl(n_pairs, table2, pid):
    p_per_w = n_pairs // _NW
    n_chunks = p_per_w // _C2
    mesh = plsc.VectorSubcoreMesh(core_axis_name="c", subcore_axis_name="s")

    @functools.partial(
        pl.kernel,
        mesh=mesh,
        out_type=jax.ShapeDtypeStruct((n_pairs, 2 * EMBEDDING_DIM), jnp.float32),
        scratch_types=[
            pltpu.VMEM((_KSUB, _SUB), jnp.int32),
            pltpu.VMEM((_C2, 2 * EMBEDDING_DIM), jnp.float32),
            pltpu.SemaphoreType.DMA,
        ],
    )
    def k(table2_hbm, pid_hbm, out_hbm, pid_v, rows_v, sem):
        wid = lax.axis_index("s") * _NC + lax.axis_index("c")
        w_base = wid * p_per_w

        def body(g, carry):
            pbase = pl.multiple_of(w_base + g * _C2, _C2)
            pltpu.sync_copy(
                pid_hbm.at[pl.ds(pl.multiple_of(pbase // _SUB, _KSUB), _KSUB)],
                pid_v,
            )
            copies = [
                pltpu.async_copy(
                    table2_hbm.at[pid_v.at[j]],
                    rows_v.at[pl.ds(j * _SUB, _SUB)],
                    sem,
                )
                for j in range(_KSUB)
            ]
            for c in copies:
                c.wait()
            pltpu.sync_copy(rows_v, out_hbm.at[pl.ds(pbase, _C2)])
            return carry

        lax.fori_loop(0, n_chunks, body, 0)

    return k(table2, pid)


def kernel(inputs, table):
    b, t = inputs.shape
    n_rows = b * t
    n_pairs = n_rows // 2
    ij = inputs.reshape(n_pairs, 2)
    pid = (ij[:, 0] * 4 + ij[:, 1]).reshape(n_pairs // _SUB, _SUB)
    table2 = jnp.concatenate(
        [jnp.repeat(table, 4, axis=0), jnp.tile(table, (4, 1))], axis=1)
    out2 = _gather_impl(n_pairs, table2, pid)
    return out2.reshape(b, t, EMBEDDING_DIM)


# speedup vs baseline: 3.8078x; 2.3427x over previous
import functools

import jax
import jax.numpy as jnp
from jax import lax
from jax.experimental import pallas as pl
from jax.experimental.pallas import tpu as pltpu
from jax.experimental.pallas import tpu_sc as plsc

EMBEDDING_DIM = 64

_info = plsc.get_sparse_core_info()
_NC, _NS = _info.num_cores, _info.num_subcores
_NW = _NC * _NS

_SUB = 128
_C2 = 512
_KSUB = _C2 // _SUB


def _gather_impl(n_pairs, table2, pid):
    p_per_w = n_pairs // _NW
    n_chunks = p_per_w // _C2
    mesh = plsc.VectorSubcoreMesh(core_axis_name="c", subcore_axis_name="s")

    @functools.partial(
        pl.kernel,
        mesh=mesh,
        out_type=jax.ShapeDtypeStruct((n_pairs, 2 * EMBEDDING_DIM), jnp.float32),
        scratch_types=[
            pltpu.VMEM((_KSUB, _SUB), jnp.int32),
            pltpu.VMEM((_C2, 2 * EMBEDDING_DIM), jnp.float32),
            pltpu.VMEM_SHARED((16, 2 * EMBEDDING_DIM), jnp.float32),
            pltpu.SemaphoreType.DMA,
        ],
    )
    def k(table2_hbm, pid_hbm, out_hbm, pid_v, rows_v, table2_sh, sem):
        sid = lax.axis_index("s")
        wid = sid * _NC + lax.axis_index("c")
        w_base = wid * p_per_w

        @pl.when(sid == 0)
        def _stage_table():
            pltpu.sync_copy(table2_hbm, table2_sh)

        plsc.subcore_barrier()

        def body(g, carry):
            pbase = pl.multiple_of(w_base + g * _C2, _C2)
            pltpu.sync_copy(
                pid_hbm.at[pl.ds(pl.multiple_of(pbase // _SUB, _KSUB), _KSUB)],
                pid_v,
            )
            copies = [
                pltpu.async_copy(
                    table2_sh.at[pid_v.at[j]],
                    rows_v.at[pl.ds(j * _SUB, _SUB)],
                    sem,
                )
                for j in range(_KSUB)
            ]
            for c in copies:
                c.wait()
            pltpu.sync_copy(rows_v, out_hbm.at[pl.ds(pbase, _C2)])
            return carry

        lax.fori_loop(0, n_chunks, body, 0)

    return k(table2, pid)


def kernel(inputs, table):
    b, t = inputs.shape
    n_rows = b * t
    n_pairs = n_rows // 2
    ij = inputs.reshape(n_pairs, 2)
    pid = (ij[:, 0] * 4 + ij[:, 1]).reshape(n_pairs // _SUB, _SUB)
    table2 = jnp.concatenate(
        [jnp.repeat(table, 4, axis=0), jnp.tile(table, (4, 1))], axis=1)
    out2 = _gather_impl(n_pairs, table2, pid)
    return out2.reshape(b, t, EMBEDDING_DIM)


# speedup vs baseline: 4.6717x; 1.2269x over previous
import functools

import jax
import jax.numpy as jnp
from jax import lax
from jax.experimental import pallas as pl
from jax.experimental.pallas import tpu as pltpu
from jax.experimental.pallas import tpu_sc as plsc

EMBEDDING_DIM = 64
_D = EMBEDDING_DIM
_L = 16

_info = plsc.get_sparse_core_info()
_NC, _NS = _info.num_cores, _info.num_subcores
_NW = _NC * _NS

_CR = 512
_UNROLL = 8


def _lookup_impl(n_rows, table, idx):
    r_per_w = n_rows // _NW
    n_chunks = r_per_w // _CR
    mesh = plsc.VectorSubcoreMesh(core_axis_name="c", subcore_axis_name="s")

    @functools.partial(
        pl.kernel,
        mesh=mesh,
        out_type=jax.ShapeDtypeStruct((n_rows, _D), jnp.float32),
        scratch_types=[
            pltpu.VMEM((4, _D), jnp.float32),
            pltpu.VMEM((_CR, _D), jnp.float32),
            pltpu.VMEM_SHARED((_NS, _CR), jnp.int32),
            pltpu.SMEM((_CR,), jnp.int32),
        ],
    )
    def k(table_hbm, idx_hbm, out_hbm, table_v, rows_v, sp_idx, sm_idx):
        sid = lax.axis_index("s")
        wid = sid * _NC + lax.axis_index("c")
        w_base = wid * r_per_w
        pltpu.sync_copy(table_hbm, table_v)

        def body(g, carry):
            base = pl.multiple_of(w_base + g * _CR, _CR)
            pltpu.sync_copy(idx_hbm.at[pl.ds(base, _CR)], sp_idx.at[sid])
            pltpu.sync_copy(sp_idx.at[sid], sm_idx)

            def row_blk(rb, carry2):
                r0 = rb * _UNROLL
                for u in range(_UNROLL):
                    r = r0 + u
                    i = sm_idx[r]
                    for c in range(_D // _L):
                        rows_v[r, pl.ds(c * _L, _L)] = (
                            table_v[i, pl.ds(c * _L, _L)])
                return carry2

            lax.fori_loop(0, _CR // _UNROLL, row_blk, 0)
            pltpu.sync_copy(rows_v, out_hbm.at[pl.ds(base, _CR)])
            return carry

        lax.fori_loop(0, n_chunks, body, 0)

    return k(table, idx)


def kernel(inputs, table):
    b, t = inputs.shape
    n_rows = b * t
    idx = inputs.reshape(n_rows)
    out = _lookup_impl(n_rows, table, idx)
    return out.reshape(b, t, _D)


# speedup vs baseline: 8.4102x; 1.8002x over previous
import functools

import jax
import jax.numpy as jnp
from jax import lax
from jax.experimental import pallas as pl
from jax.experimental.pallas import tpu as pltpu
from jax.experimental.pallas import tpu_sc as plsc

EMBEDDING_DIM = 64
_D = EMBEDDING_DIM
_L = 16

_info = plsc.get_sparse_core_info()
_NC, _NS = _info.num_cores, _info.num_subcores
_NW = _NC * _NS

_CR = 256
_UNROLL = 8


def _lookup_impl(n_rows, table, idx):
    r_per_w = n_rows // _NW
    n_chunks = r_per_w // _CR
    mesh = plsc.VectorSubcoreMesh(core_axis_name="c", subcore_axis_name="s")

    @functools.partial(
        pl.kernel,
        mesh=mesh,
        out_type=jax.ShapeDtypeStruct((n_rows, _D), jnp.float32),
        scratch_types=[
            pltpu.VMEM((4, _D), jnp.float32),
            pltpu.VMEM((_CR, _D), jnp.float32),
            pltpu.VMEM((_CR, _D), jnp.float32),
            pltpu.VMEM_SHARED((_NS, _CR), jnp.int32),
            pltpu.SMEM((_CR,), jnp.int32),
            pltpu.SemaphoreType.DMA,
            pltpu.SemaphoreType.DMA,
        ],
    )
    def k(table_hbm, idx_hbm, out_hbm, table_v, rows0, rows1, sp_idx, sm_idx,
          sem0, sem1):
        sid = lax.axis_index("s")
        wid = sid * _NC + lax.axis_index("c")
        w_base = wid * r_per_w
        pltpu.sync_copy(table_hbm, table_v)

        def chunk(g, rows_v, sem):
            base = pl.multiple_of(w_base + g * _CR, _CR)
            pltpu.sync_copy(idx_hbm.at[pl.ds(base, _CR)], sp_idx.at[sid])
            pltpu.sync_copy(sp_idx.at[sid], sm_idx)

            @pl.when(g >= 2)
            def _wait_prev():
                pltpu.make_async_copy(
                    rows_v, out_hbm.at[pl.ds(base, _CR)], sem).wait()

            def row_blk(rb, carry2):
                r0 = rb * _UNROLL
                vals = []
                for u in range(_UNROLL):
                    i = sm_idx[r0 + u]
                    for c in range(_D // _L):
                        vals.append(table_v[i, pl.ds(c * _L, _L)])
                for u in range(_UNROLL):
                    for c in range(_D // _L):
                        rows_v[r0 + u, pl.ds(c * _L, _L)] = (
                            vals[u * (_D // _L) + c])
                return carry2

            lax.fori_loop(0, _CR // _UNROLL, row_blk, 0)
            pltpu.async_copy(rows_v, out_hbm.at[pl.ds(base, _CR)], sem)

        def body(gp, carry):
            chunk(2 * gp, rows0, sem0)
            chunk(2 * gp + 1, rows1, sem1)
            return carry

        lax.fori_loop(0, n_chunks // 2, body, 0)
        pltpu.make_async_copy(
            rows0, out_hbm.at[pl.ds(w_base, _CR)], sem0).wait()
        pltpu.make_async_copy(
            rows1, out_hbm.at[pl.ds(w_base, _CR)], sem1).wait()

    return k(table, idx)


def kernel(inputs, table):
    b, t = inputs.shape
    n_rows = b * t
    idx = inputs.reshape(n_rows)
    out = _lookup_impl(n_rows, table, idx)
    return out.reshape(b, t, _D)


# speedup vs baseline: 10.3536x; 1.2311x over previous
import functools

import jax
import jax.numpy as jnp
from jax import lax
from jax.experimental import pallas as pl
from jax.experimental.pallas import tpu as pltpu
from jax.experimental.pallas import tpu_sc as plsc

EMBEDDING_DIM = 64
_D = EMBEDDING_DIM
_L = 16
_NCH = _D // _L

_info = plsc.get_sparse_core_info()
_NC, _NS = _info.num_cores, _info.num_subcores
_NW = _NC * _NS

_CR = 256
_SCR = 1024
_NSUB = _SCR // _CR
_UNROLL = 16


def _lookup_impl(n_rows, table, idx):
    r_per_w = n_rows // _NW
    n_super = r_per_w // _SCR
    mesh = plsc.VectorSubcoreMesh(core_axis_name="c", subcore_axis_name="s")

    @functools.partial(
        pl.kernel,
        mesh=mesh,
        out_type=jax.ShapeDtypeStruct((n_rows, _D), jnp.float32),
        scratch_types=[
            pltpu.VMEM((4, _D), jnp.float32),
            pltpu.VMEM((_CR, _D), jnp.float32),
            pltpu.VMEM((_CR, _D), jnp.float32),
            pltpu.VMEM_SHARED((_NS, _SCR), jnp.int32),
            pltpu.SMEM((_SCR,), jnp.int32),
            pltpu.SemaphoreType.DMA,
            pltpu.SemaphoreType.DMA,
        ],
    )
    def k(table_hbm, idx_hbm, out_hbm, table_v, rows0, rows1, sp_idx, sm_idx,
          sem0, sem1):
        sid = lax.axis_index("s")
        wid = sid * _NC + lax.axis_index("c")
        w_base = wid * r_per_w
        pltpu.sync_copy(table_hbm, table_v)

        def load_row(i):
            return [table_v[i, pl.ds(c * _L, _L)] for c in range(_NCH)]

        def store_row(rows_v, r, vals):
            for c in range(_NCH):
                rows_v[r, pl.ds(c * _L, _L)] = vals[c]

        def chunk(gs, sub, rows_v, sem):
            base = pl.multiple_of(w_base + gs * _SCR + sub * _CR, _CR)
            s0 = sub * _CR

            @pl.when(jnp.logical_or(gs >= 1, jnp.int32(sub) >= 2))
            def _wait_prev():
                pltpu.make_async_copy(
                    rows_v, out_hbm.at[pl.ds(base, _CR)], sem).wait()

            @plsc.parallel_loop(0, _CR, 1, unroll=_UNROLL)
            def _row(r):
                store_row(rows_v, r, load_row(sm_idx[s0 + r]))
            pltpu.async_copy(rows_v, out_hbm.at[pl.ds(base, _CR)], sem)

        def body(gs, carry):
            sbase = pl.multiple_of(w_base + gs * _SCR, _SCR)
            pltpu.sync_copy(idx_hbm.at[pl.ds(sbase, _SCR)], sp_idx.at[sid])
            pltpu.sync_copy(sp_idx.at[sid], sm_idx)
            for sub in range(_NSUB):
                chunk(gs, sub, (rows0, rows1)[sub % 2], (sem0, sem1)[sub % 2])
            return carry

        lax.fori_loop(0, n_super, body, 0)
        pltpu.make_async_copy(
            rows0, out_hbm.at[pl.ds(w_base, _CR)], sem0).wait()
        pltpu.make_async_copy(
            rows1, out_hbm.at[pl.ds(w_base, _CR)], sem1).wait()

    return k(table, idx)


def kernel(inputs, table):
    b, t = inputs.shape
    n_rows = b * t
    idx = inputs.reshape(n_rows)
    out = _lookup_impl(n_rows, table, idx)
    return out.reshape(b, t, _D)
